# baseline (speedup 1.0000x reference)
import functools

import jax
import jax.numpy as jnp
from jax.experimental import pallas as pl
from jax.experimental.pallas import tpu as pltpu
from jax.experimental.pallas import tpu_sc as plsc

K = 8
E = 64
L = 16
NW = 32
HB = 8


def _gate_body(x_ref, w8_ref, b8_ref, o_ref):
    C = x_ref.shape[1]
    wd = x_ref.shape[3]
    x2 = x_ref[0].reshape(C * HB, wd)
    o_ref[0, 0] = jax.lax.dot_general(
        x2, w8_ref[...], (((0,), (0,)), ((), ())),
        preferred_element_type=jnp.float32,
    ) + b8_ref[...]


def _gate_logits(x, W, b):
    B, C, H, Wd = x.shape
    w8 = jnp.einsum("hp,ce->chpe", jnp.eye(HB, dtype=W.dtype), W.T)
    w8 = w8.reshape(C * HB, HB * E)
    b8 = jnp.tile(b, (HB,)).reshape(1, HB * E)
    nj = H // HB
    logits = pl.pallas_call(
        _gate_body,
        grid=(B, nj),
        in_specs=[
            pl.BlockSpec((1, C, HB, Wd), lambda i, j: (i, 0, j, 0)),
            pl.BlockSpec((C * HB, HB * E), lambda i, j: (0, 0)),
            pl.BlockSpec((1, HB * E), lambda i, j: (0, 0)),
        ],
        out_specs=pl.BlockSpec((1, 1, Wd, HB * E), lambda i, j: (i, j, 0, 0)),
        out_shape=jax.ShapeDtypeStruct((B, nj, Wd, HB * E), jnp.float32),
    )(x, w8, b8)
    return logits.reshape(B * nj * Wd // 2, 2, HB * E)


def _lane_iota():
    return jax.lax.broadcasted_iota(jnp.int32, (L,), 0)


def _permute(v, idx):
    return jax.lax.gather(
        v, idx[:, None],
        jax.lax.GatherDimensionNumbers(
            offset_dims=(), collapsed_slice_dims=(0,), start_index_map=(0,)),
        (1,),
        mode=jax.lax.GatherScatterMode.PROMISE_IN_BOUNDS,
    )


def _merge_top8(a, b):
    ak, av = a
    bk, bv = b
    lanes = _lane_iota()
    in_a = lanes < 8
    sh = jnp.where(in_a, 0, lanes - 8)
    ck = jnp.where(in_a, ak, _permute(bk, sh))
    cv = jnp.where(in_a, av, _permute(bv, sh))
    return plsc.sort_key_val(ck, cv, descending=True)


def _topk_group(in_ref, wb_ref, ib_ref):
    lanes = _lane_iota()
    zero_idx = jnp.zeros((L,), jnp.int32)
    last_idx = jnp.full((L,), L - 1, jnp.int32)
    topk_mask = lanes < K
    for hh in range(HB):
        for dw in range(2):
            chunks = []
            for c in range(E // L):
                v = in_ref[dw, pl.ds(hh * E + c * L, L)]
                ids = lanes + (c * L)
                chunks.append(plsc.sort_key_val(v, ids, descending=True))
            m01 = _merge_top8(chunks[0], chunks[1])
            m23 = _merge_top8(chunks[2], chunks[3])
            mk, mv = _merge_top8(m01, m23)
            mx = _permute(mk, zero_idx)
            ez = jnp.exp(mk - mx)
            ez = jnp.where(topk_mask, ez, jnp.float32(0.0))
            tot = _permute(plsc.cumsum(ez), last_idx)
            wb_ref[hh, dw] = ez / tot
            ib_ref[hh, dw] = mv


def _topk_sc(logits):
    G = logits.shape[0]
    per = G // NW
    nbj = G // L
    mesh = plsc.VectorSubcoreMesh(
        core_axis_name="c", subcore_axis_name="s", num_cores=2, num_subcores=16)

    @functools.partial(
        pl.kernel,
        out_type=[
            jax.ShapeDtypeStruct((nbj, HB, 2 * L, L), jnp.float32),
            jax.ShapeDtypeStruct((nbj, HB, 2 * L, L), jnp.int32),
        ],
        mesh=mesh,
        compiler_params=pltpu.CompilerParams(needs_layout_passes=False),
        scratch_types=[
            pltpu.VMEM((2, HB * E), jnp.float32),
            pltpu.VMEM((2, HB * E), jnp.float32),
            pltpu.VMEM((HB, 2, L), jnp.float32),
            pltpu.VMEM((HB, 2, L), jnp.int32),
            pltpu.VMEM((HB, 2, L), jnp.float32),
            pltpu.VMEM((HB, 2, L), jnp.int32),
            pltpu.SemaphoreType.DMA((2,)),
            pltpu.SemaphoreType.DMA((4,)),
        ],
    )
    def run(lg_hbm, ow_hbm, oi_hbm, in0, in1, wb0, ib0, wb1, ib1, isem, osem):
        wid = jax.lax.axis_index("s") * 2 + jax.lax.axis_index("c")
        base = wid * per

        def in_copy(g, buf, slot):
            return pltpu.make_async_copy(lg_hbm.at[g], buf, isem.at[slot])

        def out_copies(g, wb, ib, so):
            bj = g // L
            w0 = 2 * (g % L)
            return (
                pltpu.make_async_copy(
                    wb, ow_hbm.at[bj, :, pl.ds(w0, 2), :], osem.at[so]),
                pltpu.make_async_copy(
                    ib, oi_hbm.at[bj, :, pl.ds(w0, 2), :], osem.at[so + 1]),
            )

        in_copy(base, in0, 0).start()

        def pair(p, carry):
            g0 = base + 2 * p
            g1 = g0 + 1
            in_copy(g1, in1, 1).start()
            in_copy(g0, in0, 0).wait()

            @pl.when(p > 0)
            def _():
                for cp in out_copies(g0 - 2, wb0, ib0, 0) + out_copies(
                        g1 - 2, wb1, ib1, 2):
                    cp.wait()

            _topk_group(in0, wb0, ib0)
            for cp in out_copies(g0, wb0, ib0, 0):
                cp.start()

            @pl.when(p + 1 < per // 2)
            def _():
                in_copy(g0 + 2, in0, 0).start()

            in_copy(g1, in1, 1).wait()
            _topk_group(in1, wb1, ib1)
            for cp in out_copies(g1, wb1, ib1, 2):
                cp.start()
            return carry

        jax.lax.fori_loop(0, per // 2, pair, 0)
        for cp in out_copies(base, wb0, ib0, 0) + out_copies(base, wb1, ib1, 2):
            cp.wait()

    return run(logits)


def kernel(x, W, b):
    B, C, H, Wd = x.shape
    logits = _gate_logits(x, W, b)
    w16, i16 = _topk_sc(logits)
    w = w16.reshape(B, H, Wd, L)[:, :, :, :K]
    i = i16.reshape(B, H, Wd, L)[:, :, :, :K]
    return w, i


# speedup vs baseline: 2.7526x; 2.7526x over previous
import functools

import jax
import jax.numpy as jnp
from jax.experimental import pallas as pl
from jax.experimental.pallas import tpu as pltpu
from jax.experimental.pallas import tpu_sc as plsc

K = 8
E = 64
L = 16
NW = 32


def _gate_body(x_ref, w_ref, b_ref, o_ref):
    o_ref[0] = jax.lax.dot_general(
        x_ref[0], w_ref[...], (((0,), (1,)), ((), ())),
        preferred_element_type=jnp.float32,
    ) + b_ref[...]


def _gate_logits(x, W, b):
    B, C, H, Wd = x.shape
    T = H * Wd
    xr = x.reshape(B, C, T)
    b2 = b.reshape(1, E)
    logits = pl.pallas_call(
        _gate_body,
        grid=(B,),
        in_specs=[
            pl.BlockSpec((1, C, T), lambda i: (i, 0, 0)),
            pl.BlockSpec((E, C), lambda i: (0, 0)),
            pl.BlockSpec((1, E), lambda i: (0, 0)),
        ],
        out_specs=pl.BlockSpec((1, T, E), lambda i: (i, 0, 0)),
        out_shape=jax.ShapeDtypeStruct((B, T, E), jnp.float32),
    )(xr, W, b2)
    return logits.reshape(B * T // L, L, E)


def _lane_iota():
    return jax.lax.broadcasted_iota(jnp.int32, (L,), 0)


def _permute(v, idx):
    return jax.lax.gather(
        v, idx[:, None],
        jax.lax.GatherDimensionNumbers(
            offset_dims=(), collapsed_slice_dims=(0,), start_index_map=(0,)),
        (1,),
        mode=jax.lax.GatherScatterMode.PROMISE_IN_BOUNDS,
    )


def _merge_top8(a, b):
    ak, av = a
    bk, bv = b
    lanes = _lane_iota()
    in_a = lanes < 8
    sh = jnp.where(in_a, 0, lanes - 8)
    ck = jnp.where(in_a, ak, _permute(bk, sh))
    cv = jnp.where(in_a, av, _permute(bv, sh))
    return plsc.sort_key_val(ck, cv, descending=True)


SG = 4


def _topk_group(in_ref, wb_ref, ib_ref):
    lanes = _lane_iota()
    zero_idx = jnp.zeros((L,), jnp.int32)
    last_idx = jnp.full((L,), L - 1, jnp.int32)
    topk_mask = lanes < K
    for gg in range(SG):
        for l in range(L):
            chunks = []
            for c in range(E // L):
                v = in_ref[gg, l, pl.ds(c * L, L)]
                ids = lanes + (c * L)
                chunks.append(plsc.sort_key_val(v, ids, descending=True))
            m01 = _merge_top8(chunks[0], chunks[1])
            m23 = _merge_top8(chunks[2], chunks[3])
            mk, mv = _merge_top8(m01, m23)
            mx = _permute(mk, zero_idx)
            ez = jnp.exp(mk - mx)
            ez = jnp.where(topk_mask, ez, jnp.float32(0.0))
            tot = _permute(plsc.cumsum(ez), last_idx)
            wb_ref[gg, l] = ez / tot
            ib_ref[gg, l] = mv


def _topk_sc(logits):
    G = logits.shape[0]
    per = G // NW // SG
    mesh = plsc.VectorSubcoreMesh(
        core_axis_name="c", subcore_axis_name="s", num_cores=2, num_subcores=16)

    @functools.partial(
        pl.kernel,
        out_type=[
            jax.ShapeDtypeStruct((G, L, L), jnp.float32),
            jax.ShapeDtypeStruct((G, L, L), jnp.int32),
        ],
        mesh=mesh,
        compiler_params=pltpu.CompilerParams(needs_layout_passes=False),
        scratch_types=[
            pltpu.VMEM((SG, L, E), jnp.float32),
            pltpu.VMEM((SG, L, E), jnp.float32),
            pltpu.VMEM((SG, L, L), jnp.float32),
            pltpu.VMEM((SG, L, L), jnp.int32),
            pltpu.VMEM((SG, L, L), jnp.float32),
            pltpu.VMEM((SG, L, L), jnp.int32),
            pltpu.SemaphoreType.DMA((2,)),
            pltpu.SemaphoreType.DMA((4,)),
        ],
    )
    def run(lg_hbm, ow_hbm, oi_hbm, in0, in1, wb0, ib0, wb1, ib1, isem, osem):
        wid = jax.lax.axis_index("s") * 2 + jax.lax.axis_index("c")
        base = wid * per

        def in_copy(g, buf, slot):
            return pltpu.make_async_copy(
                lg_hbm.at[pl.ds(g * SG, SG)], buf, isem.at[slot])

        def out_copies(g, wb, ib, so):
            return (
                pltpu.make_async_copy(
                    wb, ow_hbm.at[pl.ds(g * SG, SG)], osem.at[so]),
                pltpu.make_async_copy(
                    ib, oi_hbm.at[pl.ds(g * SG, SG)], osem.at[so + 1]),
            )

        in_copy(base, in0, 0).start()

        def pair(p, carry):
            g0 = base + 2 * p
            g1 = g0 + 1
            in_copy(g1, in1, 1).start()
            in_copy(g0, in0, 0).wait()

            @pl.when(p > 0)
            def _():
                for cp in out_copies(g0 - 2, wb0, ib0, 0) + out_copies(
                        g1 - 2, wb1, ib1, 2):
                    cp.wait()

            _topk_group(in0, wb0, ib0)
            for cp in out_copies(g0, wb0, ib0, 0):
                cp.start()

            @pl.when(p + 1 < per // 2)
            def _():
                in_copy(g0 + 2, in0, 0).start()

            in_copy(g1, in1, 1).wait()
            _topk_group(in1, wb1, ib1)
            for cp in out_copies(g1, wb1, ib1, 2):
                cp.start()
            return carry

        jax.lax.fori_loop(0, per // 2, pair, 0)
        for cp in out_copies(base, wb0, ib0, 0) + out_copies(base, wb1, ib1, 2):
            cp.wait()

    return run(logits)


def kernel(x, W, b):
    B, C, H, Wd = x.shape
    logits = _gate_logits(x, W, b)
    w16, i16 = _topk_sc(logits)
    return (w16[:, :, :K].reshape(B, H, Wd, K),
            i16[:, :, :K].reshape(B, H, Wd, K))


# speedup vs baseline: 2.9010x; 1.0539x over previous
import functools

import jax
import jax.numpy as jnp
from jax.experimental import pallas as pl
from jax.experimental.pallas import tpu as pltpu
from jax.experimental.pallas import tpu_sc as plsc

K = 8
E = 64
L = 16
NW = 32


def _gate_body(x_ref, w_ref, b_ref, o_ref):
    o_ref[0] = jax.lax.dot_general(
        x_ref[0], w_ref[...], (((0,), (1,)), ((), ())),
        preferred_element_type=jnp.float32,
    ) + b_ref[...]


def _gate_logits(x, W, b):
    B, C, H, Wd = x.shape
    T = H * Wd
    xr = x.reshape(B, C, T)
    b2 = b.reshape(1, E)
    logits = pl.pallas_call(
        _gate_body,
        grid=(B,),
        in_specs=[
            pl.BlockSpec((1, C, T), lambda i: (i, 0, 0)),
            pl.BlockSpec((E, C), lambda i: (0, 0)),
            pl.BlockSpec((1, E), lambda i: (0, 0)),
        ],
        out_specs=pl.BlockSpec((1, T, E), lambda i: (i, 0, 0)),
        out_shape=jax.ShapeDtypeStruct((B, T, E), jnp.float32),
    )(xr, W, b2)
    return logits.reshape(B * T // L, L, E)


def _lane_iota():
    return jax.lax.broadcasted_iota(jnp.int32, (L,), 0)


def _permute(v, idx):
    return jax.lax.gather(
        v, idx[:, None],
        jax.lax.GatherDimensionNumbers(
            offset_dims=(), collapsed_slice_dims=(0,), start_index_map=(0,)),
        (1,),
        mode=jax.lax.GatherScatterMode.PROMISE_IN_BOUNDS,
    )


def _merge_top8(a, b):
    ak, av = a
    bk, bv = b
    lanes = _lane_iota()
    in_a = lanes < 8
    sh = jnp.where(in_a, 0, lanes - 8)
    ck = jnp.where(in_a, ak, _permute(bk, sh))
    cv = jnp.where(in_a, av, _permute(bv, sh))
    return plsc.sort_key_val(ck, cv, descending=True)


def _topk_group(in_ref, wb_ref, ib_ref):
    lanes = _lane_iota()
    zero_idx = jnp.zeros((L,), jnp.int32)
    last_idx = jnp.full((L,), L - 1, jnp.int32)
    topk_mask = lanes < K
    for l in range(L):
        chunks = []
        for c in range(E // L):
            v = in_ref[l, pl.ds(c * L, L)]
            ids = lanes + (c * L)
            chunks.append(plsc.sort_key_val(v, ids, descending=True))
        m01 = _merge_top8(chunks[0], chunks[1])
        m23 = _merge_top8(chunks[2], chunks[3])
        mk, mv = _merge_top8(m01, m23)
        mx = _permute(mk, zero_idx)
        ez = jnp.exp(mk - mx)
        ez = jnp.where(topk_mask, ez, jnp.float32(0.0))
        tot = _permute(plsc.cumsum(ez), last_idx)
        wb_ref[l] = ez / tot
        ib_ref[l] = mv


def _topk_sc(logits):
    G = logits.shape[0]
    per = G // NW
    mesh = plsc.VectorSubcoreMesh(
        core_axis_name="c", subcore_axis_name="s", num_cores=2, num_subcores=16)

    @functools.partial(
        pl.kernel,
        out_type=[
            jax.ShapeDtypeStruct((G, L, L), jnp.float32),
            jax.ShapeDtypeStruct((G, L, L), jnp.int32),
        ],
        mesh=mesh,
        compiler_params=pltpu.CompilerParams(needs_layout_passes=False),
        scratch_types=[
            pltpu.VMEM((L, E), jnp.float32),
            pltpu.VMEM((L, E), jnp.float32),
            pltpu.VMEM((L, L), jnp.float32),
            pltpu.VMEM((L, L), jnp.int32),
            pltpu.VMEM((L, L), jnp.float32),
            pltpu.VMEM((L, L), jnp.int32),
            pltpu.SemaphoreType.DMA((2,)),
            pltpu.SemaphoreType.DMA((4,)),
        ],
    )
    def run(lg_hbm, ow_hbm, oi_hbm, in0, in1, wb0, ib0, wb1, ib1, isem, osem):
        wid = jax.lax.axis_index("s") * 2 + jax.lax.axis_index("c")
        base = wid * per

        def in_copy(g, buf, slot):
            return pltpu.make_async_copy(lg_hbm.at[g], buf, isem.at[slot])

        def out_copies(g, wb, ib, so):
            return (
                pltpu.make_async_copy(wb, ow_hbm.at[g], osem.at[so]),
                pltpu.make_async_copy(ib, oi_hbm.at[g], osem.at[so + 1]),
            )

        in_copy(base, in0, 0).start()

        def pair(p, carry):
            g0 = base + 2 * p
            g1 = g0 + 1
            in_copy(g1, in1, 1).start()
            in_copy(g0, in0, 0).wait()

            @pl.when(p > 0)
            def _():
                for cp in out_copies(g0 - 2, wb0, ib0, 0) + out_copies(
                        g1 - 2, wb1, ib1, 2):
                    cp.wait()

            _topk_group(in0, wb0, ib0)
            for cp in out_copies(g0, wb0, ib0, 0):
                cp.start()

            @pl.when(p + 1 < per // 2)
            def _():
                in_copy(g0 + 2, in0, 0).start()

            in_copy(g1, in1, 1).wait()
            _topk_group(in1, wb1, ib1)
            for cp in out_copies(g1, wb1, ib1, 2):
                cp.start()
            return carry

        jax.lax.fori_loop(0, per // 2, pair, 0)
        for cp in out_copies(base, wb0, ib0, 0) + out_copies(base, wb1, ib1, 2):
            cp.wait()

    return run(logits)


def kernel(x, W, b):
    B, C, H, Wd = x.shape
    logits = _gate_logits(x, W, b)
    w16, i16 = _topk_sc(logits)
    return (w16[:, :, :K].reshape(B, H, Wd, K),
            i16[:, :, :K].reshape(B, H, Wd, K))
